# Initial kernel scaffold; baseline (speedup 1.0000x reference)
import functools

import jax
import jax.numpy as jnp
from jax import lax
from jax.experimental import pallas as pl
from jax.experimental.pallas import tpu as pltpu
from jax.experimental.pallas import tpu_sc as plsc

N = 320000
D = 128
H = 64
M = 10000

NC = 2
NS = 16
NW = NC * NS
LANES = 16

CHUNK = N // NW
VECS = CHUNK // LANES
M_PAD = 10240

BN = 4000


def _mlp_body(x_ref, w1_ref, b1_ref, w2_ref, b2_ref, y_ref):
    x = x_ref[...]
    h = jnp.dot(x, w1_ref[...], preferred_element_type=jnp.float32)
    h = h + b1_ref[...]
    h = h * jax.nn.sigmoid(h)
    y = jnp.sum(h * w2_ref[...], axis=1) + b2_ref[0]
    y_ref[...] = y


def _mlp(x, W1, b1, w2row, b2):
    return pl.pallas_call(
        _mlp_body,
        grid=(N // BN,),
        in_specs=[
            pl.BlockSpec((BN, D), lambda i: (i, 0)),
            pl.BlockSpec((D, H), lambda i: (0, 0)),
            pl.BlockSpec((H,), lambda i: (0,)),
            pl.BlockSpec((1, H), lambda i: (0, 0)),
            pl.BlockSpec(memory_space=pltpu.SMEM),
        ],
        out_specs=pl.BlockSpec((BN,), lambda i: (i,)),
        out_shape=jax.ShapeDtypeStruct((N,), jnp.float32),
    )(x, W1, b1, w2row, b2)


def _seg_body(y_hbm, idx_hbm, part_hbm, idx_v, y_v, acc_v):
    wid = lax.axis_index("s") * NC + lax.axis_index("c")
    base = wid * CHUNK

    pltpu.sync_copy(idx_hbm.at[pl.ds(base, CHUNK)], idx_v.at[pl.ds(LANES, CHUNK)])
    pltpu.sync_copy(y_hbm.at[pl.ds(base, CHUNK)], y_v)

    sentinel = jnp.full((LANES,), -1, jnp.int32)
    idx_v[pl.ds(0, LANES)] = sentinel
    idx_v[pl.ds(LANES + CHUNK, LANES)] = sentinel

    def zero_body(i, c):
        acc_v[pl.ds(i * LANES, LANES)] = jnp.zeros((LANES,), jnp.float32)
        return c

    lax.fori_loop(0, M_PAD // LANES, zero_body, 0)

    lanes = lax.iota(jnp.int32, LANES)

    def body(t, carry):
        b = LANES + t * LANES
        iv = idx_v[pl.ds(b, LANES)]
        prev = plsc.load_gather(idx_v, [lanes + (b - 1)])
        nxt = plsc.load_gather(idx_v, [lanes + (b + 1)])
        yv = y_v[pl.ds(t * LANES, LANES)]
        cs = plsc.cumsum(yv) + carry
        cex = cs - yv
        plsc.addupdate_scatter(acc_v, [iv], cs, mask=iv != nxt)
        plsc.addupdate_scatter(acc_v, [iv], -cex, mask=iv != prev)
        return carry + jnp.sum(yv)

    lax.fori_loop(0, VECS, body, jnp.float32(0.0))

    pltpu.sync_copy(acc_v, part_hbm.at[wid])


_seg_sum = functools.partial(
    pl.kernel,
    out_type=jax.ShapeDtypeStruct((NW, M_PAD), jnp.float32),
    mesh=plsc.VectorSubcoreMesh(core_axis_name="c", subcore_axis_name="s"),
    scratch_types=[
        pltpu.VMEM((CHUNK + 2 * LANES,), jnp.int32),
        pltpu.VMEM((CHUNK,), jnp.float32),
        pltpu.VMEM((M_PAD,), jnp.float32),
    ],
)(_seg_body)


def _reduce_body(p_ref, o_ref):
    o_ref[...] = jnp.sum(p_ref[...], axis=0)


def _reduce(parts):
    return pl.pallas_call(
        _reduce_body,
        out_shape=jax.ShapeDtypeStruct((M_PAD,), jnp.float32),
    )(parts)


def kernel(scalar_representation, idx_m, W1, b1, W2, b2):
    idx = idx_m.astype(jnp.int32)
    y = _mlp(scalar_representation, W1, b1, W2.reshape(1, H), b2)
    parts = _seg_sum(y, idx)
    out = _reduce(parts)
    return out[:M]


# speedup vs baseline: 4.3898x; 4.3898x over previous
import functools

import jax
import jax.numpy as jnp
from jax import lax
from jax.experimental import pallas as pl
from jax.experimental.pallas import tpu as pltpu
from jax.experimental.pallas import tpu_sc as plsc

N = 320000
D = 128
H = 64
M = 10000

NC = 2
NS = 16
NW = NC * NS
LANES = 16

M_PAD = 10240
BN = 16384

SLABS = ((0, 163840), (163840, 156160))


def _mlp_body(x_ref, w1_ref, b1_ref, w2_ref, b2_ref, y_ref):
    x = x_ref[...]
    h = jnp.dot(x, w1_ref[...], preferred_element_type=jnp.float32)
    h = h + b1_ref[...]
    h = h * jax.nn.sigmoid(h)
    y2d = lax.dot_general(
        w2_ref[...], h, (((1,), (1,)), ((), ())),
        preferred_element_type=jnp.float32,
    )
    y_ref[...] = y2d[0] + b2_ref[0]


def _make_mlp(start, size):
    first = start // BN

    def call(x, W1, b1, w2row, b2):
        return pl.pallas_call(
            _mlp_body,
            grid=(pl.cdiv(size, BN),),
            in_specs=[
                pl.BlockSpec((BN, D), lambda i: (first + i, 0)),
                pl.BlockSpec((D, H), lambda i: (0, 0)),
                pl.BlockSpec((H,), lambda i: (0,)),
                pl.BlockSpec((1, H), lambda i: (0, 0)),
                pl.BlockSpec(memory_space=pltpu.SMEM),
            ],
            out_specs=pl.BlockSpec((BN,), lambda i: (i,)),
            out_shape=jax.ShapeDtypeStruct((size,), jnp.float32),
        )(x, W1, b1, w2row, b2)

    return call


def _make_seg(start, size):
    chunk = size // NW
    vecs = chunk // LANES

    def body(y_hbm, idx_hbm, part_hbm, idx_v, y_v, acc_v):
        wid = lax.axis_index("s") * NC + lax.axis_index("c")
        base = wid * chunk

        pltpu.sync_copy(idx_hbm.at[pl.ds(start + base, chunk)], idx_v)
        pltpu.sync_copy(y_hbm.at[pl.ds(base, chunk)], y_v)

        def zero_body(i, c):
            acc_v[pl.ds(i * LANES, LANES)] = jnp.zeros((LANES,), jnp.float32)
            return c

        lax.fori_loop(0, M_PAD // LANES, zero_body, 0)

        def seg(t, c):
            b = t * LANES
            iv = idx_v[pl.ds(b, LANES)]
            yv = y_v[pl.ds(b, LANES)]
            plsc.addupdate_scatter(acc_v, [iv], yv)
            return c

        lax.fori_loop(0, vecs, seg, 0)

        pltpu.sync_copy(acc_v, part_hbm.at[wid])

    return functools.partial(
        pl.kernel,
        out_type=jax.ShapeDtypeStruct((NW, M_PAD), jnp.float32),
        mesh=plsc.VectorSubcoreMesh(core_axis_name="c", subcore_axis_name="s"),
        compiler_params=pltpu.CompilerParams(needs_layout_passes=False),
        scratch_types=[
            pltpu.VMEM((chunk,), jnp.int32),
            pltpu.VMEM((chunk,), jnp.float32),
            pltpu.VMEM((M_PAD,), jnp.float32),
        ],
    )(body)


_MLPS = tuple(_make_mlp(s, z) for s, z in SLABS)
_SEGS = tuple(_make_seg(s, z) for s, z in SLABS)


def _reduce_body(p1_ref, p2_ref, o_ref):
    o_ref[...] = jnp.sum(p1_ref[...], axis=0) + jnp.sum(p2_ref[...], axis=0)


def _reduce(p1, p2):
    return pl.pallas_call(
        _reduce_body,
        out_shape=jax.ShapeDtypeStruct((M_PAD,), jnp.float32),
    )(p1, p2)


def kernel(scalar_representation, idx_m, W1, b1, W2, b2):
    idx = idx_m.astype(jnp.int32)
    w2row = W2.reshape(1, H)
    ys = [mlp(scalar_representation, W1, b1, w2row, b2) for mlp in _MLPS]
    parts = [seg(y, idx) for seg, y in zip(_SEGS, ys)]
    out = _reduce(*parts)
    return out[:M]
